# baseline (speedup 1.0000x reference)
import functools

import jax
import jax.numpy as jnp
from jax import lax
from jax.experimental import pallas as pl
from jax.experimental.pallas import tpu as pltpu
from jax.experimental.pallas import tpu_sc as plsc

N_VOCAB = 100000
D = 64
BATCH = 1024
SEQ = 200
OUT_SEQ = SEQ + 1
NC = 2
NS = 16
NW = NC * NS
SP = (OUT_SEQ + 1) // 2

NB = 4
CB = BATCH // NB
BC = 64
UPS = CB // BC
CUNITS = SEQ * UPS
UPW = CUNITS // NW
GRP = 5
GPW = UPW // GRP
MID_ROWS = SP * CB
BPW = CB // NW

_mesh = plsc.VectorSubcoreMesh(core_axis_name="c", subcore_axis_name="s")


@functools.partial(
    pl.kernel,
    mesh=_mesh,
    out_type=jax.ShapeDtypeStruct((MID_ROWS, 2 * D), jnp.float32),
    scratch_types=[
        pltpu.VMEM((UPW, BC), jnp.int32),
        pltpu.VMEM((GRP * BC, D), jnp.float32),
        pltpu.VMEM((GRP * BC, D), jnp.float32),
        pltpu.VMEM((1, 2 * NS), jnp.int32),
        pltpu.VMEM((BPW, D), jnp.float32),
        pltpu.SemaphoreType.DMA,
        pltpu.SemaphoreType.DMA,
    ],
    compiler_params=pltpu.CompilerParams(use_tc_tiling_on_sc=False),
)
def _embed_gather(ids_hbm, table_hbm, out_hbm, idx_v, buf0, buf1, zidx, bosb, sem0, sem1):
    wid = lax.axis_index("s") * NC + lax.axis_index("c")
    u_base = wid * UPW

    pltpu.sync_copy(ids_hbm.at[pl.ds(u_base, UPW)], idx_v)

    zidx[0, pl.ds(0, 16)] = jnp.zeros((16,), jnp.int32)
    zidx[0, pl.ds(16, 16)] = jnp.zeros((16,), jnp.int32)
    pltpu.async_copy(table_hbm.at[zidx.at[0, pl.ds(0, BPW)]], bosb, sem0).wait()
    pltpu.sync_copy(bosb, out_hbm.at[pl.ds(wid * BPW, BPW), pl.ds(0, D)])

    def issue_group(g, buf, sem):
        for j in range(GRP):
            dst = buf.at[pl.ds(j * BC, BC)]
            pltpu.async_copy(table_hbm.at[idx_v.at[g * GRP + j]], dst, sem)

    def drain_group(buf, sem):
        pltpu.make_async_copy(table_hbm.at[pl.ds(0, GRP * BC)], buf, sem).wait()

    def write_group(g, buf):
        for j in range(GRP):
            u = u_base + g * GRP + j
            s = 1 + u // UPS
            row0 = (s // 2) * CB + (u % UPS) * BC
            dst = out_hbm.at[pl.ds(row0, BC), pl.ds((s % 2) * D, D)]
            pltpu.sync_copy(buf.at[pl.ds(j * BC, BC)], dst)

    issue_group(0, buf0, sem0)

    def body(i, carry):
        g = 2 * i
        issue_group(g + 1, buf1, sem1)
        drain_group(buf0, sem0)
        write_group(g, buf0)

        @pl.when(i < GPW // 2)
        def _():
            issue_group(g + 2, buf0, sem0)

        drain_group(buf1, sem1)
        write_group(g + 1, buf1)
        return carry

    lax.fori_loop(0, GPW // 2, body, 0)
    drain_group(buf0, sem0)
    write_group(GPW - 1, buf0)


def _table_body(xa_ref, xb_ref, y_ref):
    y_ref[...] = jnp.concatenate([xa_ref[...], xb_ref[...]], axis=0).transpose()


PAIR = 49 * 1024

_tc_table = pl.pallas_call(
    _table_body,
    grid=(49,),
    in_specs=[
        pl.BlockSpec((D, 1024), lambda i: (0, i)),
        pl.BlockSpec((D, 1024), lambda i: (0, i + 49)),
    ],
    out_specs=pl.BlockSpec((1024, 2 * D), lambda i: (i, 0)),
    out_shape=jax.ShapeDtypeStruct((PAIR, 2 * D), jnp.float32),
)


def _transpose_block(x_ref, y_ref):
    xt = x_ref[...].transpose()
    y_ref[0] = xt[:D]
    y_ref[1] = xt[D:]


def _transpose_body(x_ref, y_ref):
    _transpose_block(x_ref, y_ref)


def _transpose_body_aliased(x_ref, yin_ref, y_ref):
    del yin_ref
    _transpose_block(x_ref, y_ref)


def _make_tc_transpose(c, aliased):
    return pl.pallas_call(
        _transpose_body_aliased if aliased else _transpose_body,
        grid=(SP,),
        in_specs=[pl.BlockSpec((CB, 2 * D), lambda i: (i, 0))]
        + ([pl.BlockSpec(memory_space=pl.ANY)] if aliased else []),
        out_specs=pl.BlockSpec((2, D, CB), lambda i: (i, 0, c)),
        out_shape=jax.ShapeDtypeStruct((OUT_SEQ, D, BATCH), jnp.float32),
        input_output_aliases={1: 0} if aliased else {},
    )


def kernel(input_ids, embedding_weight):
    ids_v = jnp.where(input_ids < PAIR, 2 * input_ids, 2 * input_ids - (2 * PAIR - 1))
    idsr = ids_v.T.reshape(SEQ, NB, CB).transpose(1, 0, 2)
    xt = embedding_weight.T
    wt_lin = _tc_table(xt, xt).reshape(2 * PAIR, D)
    y = None
    for c in range(NB):
        ids_c = idsr[c].reshape(CUNITS, BC)
        mid_c = _embed_gather(ids_c, wt_lin)
        tc = _make_tc_transpose(c, aliased=c > 0)
        y = tc(mid_c) if c == 0 else tc(mid_c, y)
    return jnp.transpose(y, (2, 0, 1))
